# baseline (speedup 1.0000x reference)
import functools

import jax
import jax.numpy as jnp
from jax import lax
from jax.experimental import pallas as pl
from jax.experimental.pallas import tpu as pltpu, tpu_sc as plsc

N = 10000
D = 64
G = 256
NC = 2
NS = 16
NW = NC * NS
BINS_H = 10240
ACC = 12000
SENT = N
CH = 128
BLK = 1000
ROWS_PT = ACC // NS
BPT = BINS_H // NW

EPB_P = 4096
EPB_R = 40960
EPW_P = 2048
EPW_R = 20480
NCH_P = EPW_P // CH
NCH_R = EPW_R // CH
KG = 2

def _mesh():
    return plsc.VectorSubcoreMesh(core_axis_name="c", subcore_axis_name="s")


_sc_params = pltpu.CompilerParams(needs_layout_passes=False)


def _pad_edges(ei, e_pad):
    src = ei[0].astype(jnp.int32)
    dst = ei[1].astype(jnp.int32)
    e = src.shape[0]
    src = jnp.concatenate([src, jnp.zeros((e_pad - e,), jnp.int32)])
    dst = jnp.concatenate([dst, jnp.full((e_pad - e,), SENT, jnp.int32)])
    return src, dst


def _deg_body(dstp_hbm, dstr_hbm, outp_hbm, outr_hbm,
              idx_v, hist_v, row_v, deg_v, hists_sh):
    c = lax.axis_index("c")
    s = lax.axis_index("s")
    ones = jnp.ones((16,), jnp.float32)
    zeros16 = jnp.zeros((16,), jnp.float32)
    sumbase = (c * NS + s) * BPT

    for dst_hbm, out_hbm, epb in ((dstp_hbm, outp_hbm, EPB_P),
                                  (dstr_hbm, outr_hbm, EPB_R)):
        def zb(i, _):
            hist_v[pl.ds(i * 16, 16)] = zeros16
            return ()
        lax.fori_loop(0, BINS_H // 16, zb, ())
        pltpu.sync_copy(dst_hbm.at[pl.ds(s * epb, epb)], idx_v.at[pl.ds(0, epb)])
        def hb(i, _):
            idx = idx_v[pl.ds(i * 16, 16)]
            plsc.addupdate_scatter(hist_v, [idx], ones)
            return ()
        lax.fori_loop(0, epb // 16, hb, ())
        pltpu.sync_copy(hist_v, hists_sh.at[pl.ds(s * BINS_H, BINS_H)])
        plsc.subcore_barrier()
        def db(i, _):
            deg_v[pl.ds(i * 16, 16)] = ones
            return ()
        lax.fori_loop(0, BPT // 16, db, ())
        for t in range(NS):
            pltpu.sync_copy(hists_sh.at[pl.ds(t * BINS_H + sumbase, BPT)], row_v)
            def ab(i, _):
                deg_v[pl.ds(i * 16, 16)] += row_v[pl.ds(i * 16, 16)]
                return ()
            lax.fori_loop(0, BPT // 16, ab, ())
        pltpu.sync_copy(deg_v, out_hbm.at[pl.ds(sumbase, BPT)])
        plsc.subcore_barrier()


def _sc_deg(dst_p, dst_r):
    k = pl.kernel(
        _deg_body,
        mesh=_mesh(),
        compiler_params=_sc_params,
        out_type=(jax.ShapeDtypeStruct((BINS_H,), jnp.float32),
                  jax.ShapeDtypeStruct((BINS_H,), jnp.float32)),
        scratch_types=[
            pltpu.VMEM((EPB_R,), jnp.int32),
            pltpu.VMEM((BINS_H,), jnp.float32),
            pltpu.VMEM((BPT,), jnp.float32),
            pltpu.VMEM((BPT,), jnp.float32),
            pltpu.MemorySpace.VMEM_SHARED((NS * BINS_H,), jnp.float32),
        ],
    )
    return k(dst_p, dst_r)


def _mm_body(x_ref, w_ref, deg_ref, o_ref):
    h = jnp.dot(x_ref[...], w_ref[...], preferred_element_type=jnp.float32)
    o_ref[...] = h * lax.rsqrt(deg_ref[...])


def _tc_matmul_scale(x, w, deg_col):
    f = x.shape[1]
    return pl.pallas_call(
        _mm_body,
        grid=(N // BLK,),
        in_specs=[
            pl.BlockSpec((BLK, f), lambda i: (i, 0)),
            pl.BlockSpec((f, D), lambda i: (0, 0)),
            pl.BlockSpec((BLK, 1), lambda i: (i, 0)),
        ],
        out_specs=pl.BlockSpec((BLK, D), lambda i: (i, 0)),
        out_shape=jax.ShapeDtypeStruct((N, D), jnp.float32),
    )(x, w, deg_col)


def _edges_body(hsp_hbm, srcp_hbm, dstp_hbm, hsr_hbm, srcr_hbm, dstr_hbm,
                zeros_hbm, outp_hbm, outr_hbm,
                sidx_v, didx_v, r0, r1, r2, r3,
                acc_sh, gsa, gsb, ssa, ssb):
    c = lax.axis_index("c")
    s = lax.axis_index("s")
    wid = c * NS + s
    rows = ((r0, r1), (r2, r3))
    gsem = (gsa, gsb)
    ssem = (ssa, ssb)

    for hs_hbm, src_hbm, dst_hbm, out_hbm, nch in (
            (hsp_hbm, srcp_hbm, dstp_hbm, outp_hbm, NCH_P),
            (hsr_hbm, srcr_hbm, dstr_hbm, outr_hbm, NCH_R)):
        ngrp = nch // KG
        pltpu.sync_copy(zeros_hbm, acc_sh.at[pl.ds(s * ROWS_PT, ROWS_PT)])
        pltpu.sync_copy(src_hbm.at[wid, pl.ds(0, nch)], sidx_v.at[pl.ds(0, nch)])
        pltpu.sync_copy(dst_hbm.at[wid, pl.ds(0, nch)], didx_v.at[pl.ds(0, nch)])
        plsc.subcore_barrier()

        def ch(j, _):
            pltpu.async_copy(hs_hbm.at[sidx_v.at[j]], r0, gsa).wait()
            pltpu.sync_copy(r0, acc_sh.at[didx_v.at[j]], add=True)
            return ()
        lax.fori_loop(0, nch, ch, ())
        plsc.subcore_barrier()
        pltpu.sync_copy(acc_sh.at[pl.ds(s * ROWS_PT, ROWS_PT)],
                        out_hbm.at[c, pl.ds(s * ROWS_PT, ROWS_PT)])
        plsc.subcore_barrier()


def _sc_edges(hs_p, src_p, dst_p, hs_r, src_r, dst_r):
    zeros = jnp.zeros((ROWS_PT, D), jnp.float32)
    rows_bufs = [pltpu.VMEM((CH, D), jnp.float32) for _ in range(2 * KG)]
    k = pl.kernel(
        _edges_body,
        mesh=_mesh(),
        compiler_params=pltpu.CompilerParams(
            needs_layout_passes=False, use_tc_tiling_on_sc=False),
        out_type=(jax.ShapeDtypeStruct((NC, ACC, D), jnp.float32),
                  jax.ShapeDtypeStruct((NC, ACC, D), jnp.float32)),
        scratch_types=[
            pltpu.VMEM((NCH_R, CH), jnp.int32),
            pltpu.VMEM((NCH_R, CH), jnp.int32),
            *rows_bufs,
            pltpu.MemorySpace.VMEM_SHARED((ACC, D), jnp.float32),
            pltpu.SemaphoreType.DMA,
            pltpu.SemaphoreType.DMA,
            pltpu.SemaphoreType.DMA,
            pltpu.SemaphoreType.DMA,
        ],
    )
    return k(hs_p, src_p, dst_p, hs_r, src_r, dst_r, zeros)


def _final_body(accp_ref, hsp_ref, degp_ref, ohp_ref, bp_ref,
                accr_ref, hsr_ref, degr_ref, ohr_ref, br_ref,
                lw_ref, lb_ref, out_ref, poolp, poolr):
    i = pl.program_id(0)

    @pl.when(i == 0)
    def _():
        poolp[...] = jnp.zeros_like(poolp)
        poolr[...] = jnp.zeros_like(poolr)

    def branch(acc_ref, hs_ref, deg_ref, oh_ref, b_ref, pool_ref):
        a = acc_ref[...]
        hs = hs_ref[...]
        dinv = lax.rsqrt(deg_ref[...])
        node = (a[0] + a[1] + hs) * dinv + b_ref[...]
        node = jnp.maximum(node, 0.0)
        aug = jnp.concatenate([node, jnp.ones_like(node)], axis=1)
        pool_ref[...] += lax.dot_general(
            oh_ref[...], aug, (((0,), (0,)), ((), ())),
            preferred_element_type=jnp.float32)

    branch(accp_ref, hsp_ref, degp_ref, ohp_ref, bp_ref, poolp)
    branch(accr_ref, hsr_ref, degr_ref, ohr_ref, br_ref, poolr)

    @pl.when(i == pl.num_programs(0) - 1)
    def _():
        pp = poolp[...]
        pr = poolr[...]
        mp = pp[:, :D] / jnp.maximum(pp[:, D:D + 1], 1.0)
        mr = pr[:, :D] / jnp.maximum(pr[:, D:D + 1], 1.0)
        feat = jnp.concatenate([mp, mr], axis=1)
        out_ref[...] = (jnp.dot(feat, lw_ref[...],
                                preferred_element_type=jnp.float32)
                        + lb_ref[...])


def _tc_final(accp, hs_p, degp, ohp, b_p, accr, hs_r, degr, ohr, b_r,
              lin_W, lin_b):
    return pl.pallas_call(
        _final_body,
        grid=(N // BLK,),
        in_specs=[
            pl.BlockSpec((NC, BLK, D), lambda i: (0, i, 0)),
            pl.BlockSpec((BLK, D), lambda i: (i, 0)),
            pl.BlockSpec((BLK, 1), lambda i: (i, 0)),
            pl.BlockSpec((BLK, G), lambda i: (i, 0)),
            pl.BlockSpec((1, D), lambda i: (0, 0)),
            pl.BlockSpec((NC, BLK, D), lambda i: (0, i, 0)),
            pl.BlockSpec((BLK, D), lambda i: (i, 0)),
            pl.BlockSpec((BLK, 1), lambda i: (i, 0)),
            pl.BlockSpec((BLK, G), lambda i: (i, 0)),
            pl.BlockSpec((1, D), lambda i: (0, 0)),
            pl.BlockSpec((D * 2, 2), lambda i: (0, 0)),
            pl.BlockSpec((1, 2), lambda i: (0, 0)),
        ],
        out_specs=pl.BlockSpec((G, 2), lambda i: (0, 0)),
        out_shape=jax.ShapeDtypeStruct((G, 2), jnp.float32),
        scratch_shapes=[
            pltpu.VMEM((G, 2 * D), jnp.float32),
            pltpu.VMEM((G, 2 * D), jnp.float32),
        ],
    )(accp, hs_p, degp, ohp, b_p, accr, hs_r, degr, ohr, b_r, lin_W, lin_b)


def _deg_to_col(deg):
    return jnp.concatenate(
        [deg, jnp.ones((ACC - BINS_H,), jnp.float32)]).reshape(ACC, 1)


def kernel(p_node_feat, p_edge_index, p_batch, r_node_feat, r_edge_index,
           r_batch, W_p, b_p, W_r, b_r, lin_W, lin_b):
    src_p, dst_p = _pad_edges(p_edge_index, NS * EPB_P)
    src_r, dst_r = _pad_edges(r_edge_index, NS * EPB_R)
    src_p3 = src_p.reshape(NW, NCH_P, CH)
    dst_p3 = dst_p.reshape(NW, NCH_P, CH)
    src_r3 = src_r.reshape(NW, NCH_R, CH)
    dst_r3 = dst_r.reshape(NW, NCH_R, CH)

    degp, degr = _sc_deg(dst_p, dst_r)
    degp_col = _deg_to_col(degp)
    degr_col = _deg_to_col(degr)

    hs_p = _tc_matmul_scale(p_node_feat.astype(jnp.float32), W_p, degp_col)
    hs_r = _tc_matmul_scale(r_node_feat.astype(jnp.float32), W_r, degr_col)

    accp, accr = _sc_edges(hs_p, src_p3, dst_p3, hs_r, src_r3, dst_r3)

    gids = jnp.arange(G, dtype=jnp.int32)
    ohp = (p_batch.astype(jnp.int32)[:, None] == gids[None, :]).astype(jnp.float32)
    ohr = (r_batch.astype(jnp.int32)[:, None] == gids[None, :]).astype(jnp.float32)

    return _tc_final(accp, hs_p, degp_col, ohp, b_p.reshape(1, D),
                     accr, hs_r, degr_col, ohr, b_r.reshape(1, D),
                     lin_W, lin_b.reshape(1, 2))


# speedup vs baseline: 1.0970x; 1.0970x over previous
import functools

import jax
import jax.numpy as jnp
from jax import lax
from jax.experimental import pallas as pl
from jax.experimental.pallas import tpu as pltpu, tpu_sc as plsc

N = 10000
D = 64
G = 256
NC = 2
NS = 16
NW = NC * NS
BINS_H = 10240
ACC = 12000
SENT = N
CH = 128
BLK = 1000
ROWS_PT = ACC // NS
BPT = BINS_H // NW

EPB_P = 4096
EPB_R = 40960
EPW_P = 2048
EPW_R = 20480
NCH_P = EPW_P // CH
NCH_R = EPW_R // CH
KG = 2

def _mesh():
    return plsc.VectorSubcoreMesh(core_axis_name="c", subcore_axis_name="s")


_sc_params = pltpu.CompilerParams(needs_layout_passes=False)


def _pad_edges(ei, e_pad):
    src = ei[0].astype(jnp.int32)
    dst = ei[1].astype(jnp.int32)
    e = src.shape[0]
    src = jnp.concatenate([src, jnp.zeros((e_pad - e,), jnp.int32)])
    dst = jnp.concatenate([dst, jnp.full((e_pad - e,), SENT, jnp.int32)])
    return src, dst


def _deg_body(dstp_hbm, dstr_hbm, outp_hbm, outr_hbm,
              idx_v, hist_v, row_v, deg_v, hists_sh):
    c = lax.axis_index("c")
    s = lax.axis_index("s")
    ones = jnp.ones((16,), jnp.float32)
    zeros16 = jnp.zeros((16,), jnp.float32)
    sumbase = (c * NS + s) * BPT

    for dst_hbm, out_hbm, epb in ((dstp_hbm, outp_hbm, EPB_P),
                                  (dstr_hbm, outr_hbm, EPB_R)):
        def zb(i, _):
            hist_v[pl.ds(i * 16, 16)] = zeros16
            return ()
        lax.fori_loop(0, BINS_H // 16, zb, ())
        pltpu.sync_copy(dst_hbm.at[pl.ds(s * epb, epb)], idx_v.at[pl.ds(0, epb)])
        def hb(i, _):
            idx = idx_v[pl.ds(i * 16, 16)]
            plsc.addupdate_scatter(hist_v, [idx], ones)
            return ()
        lax.fori_loop(0, epb // 16, hb, ())
        pltpu.sync_copy(hist_v, hists_sh.at[pl.ds(s * BINS_H, BINS_H)])
        plsc.subcore_barrier()
        def db(i, _):
            deg_v[pl.ds(i * 16, 16)] = ones
            return ()
        lax.fori_loop(0, BPT // 16, db, ())
        for t in range(NS):
            pltpu.sync_copy(hists_sh.at[pl.ds(t * BINS_H + sumbase, BPT)], row_v)
            def ab(i, _):
                deg_v[pl.ds(i * 16, 16)] += row_v[pl.ds(i * 16, 16)]
                return ()
            lax.fori_loop(0, BPT // 16, ab, ())
        pltpu.sync_copy(deg_v, out_hbm.at[pl.ds(sumbase, BPT)])
        plsc.subcore_barrier()


def _sc_deg(dst_p, dst_r):
    k = pl.kernel(
        _deg_body,
        mesh=_mesh(),
        compiler_params=_sc_params,
        out_type=(jax.ShapeDtypeStruct((BINS_H,), jnp.float32),
                  jax.ShapeDtypeStruct((BINS_H,), jnp.float32)),
        scratch_types=[
            pltpu.VMEM((EPB_R,), jnp.int32),
            pltpu.VMEM((BINS_H,), jnp.float32),
            pltpu.VMEM((BPT,), jnp.float32),
            pltpu.VMEM((BPT,), jnp.float32),
            pltpu.MemorySpace.VMEM_SHARED((NS * BINS_H,), jnp.float32),
        ],
    )
    return k(dst_p, dst_r)


def _mm_body(x_ref, w_ref, deg_ref, o_ref):
    h = jnp.dot(x_ref[...], w_ref[...], preferred_element_type=jnp.float32)
    o_ref[...] = h * lax.rsqrt(deg_ref[...])


def _tc_matmul_scale(x, w, deg_col):
    f = x.shape[1]
    return pl.pallas_call(
        _mm_body,
        grid=(N // BLK,),
        in_specs=[
            pl.BlockSpec((BLK, f), lambda i: (i, 0)),
            pl.BlockSpec((f, D), lambda i: (0, 0)),
            pl.BlockSpec((BLK, 1), lambda i: (i, 0)),
        ],
        out_specs=pl.BlockSpec((BLK, D), lambda i: (i, 0)),
        out_shape=jax.ShapeDtypeStruct((N, D), jnp.float32),
    )(x, w, deg_col)


NB = 4


def _edges_body(hsp_hbm, srcp_hbm, dstp_hbm, hsr_hbm, srcr_hbm, dstr_hbm,
                zeros_hbm, outp_hbm, outr_hbm,
                si0, si1, si2, si3, di0, di1, di2, di3,
                r0, r1, r2, r3, acc_sh, g0, g1, g2, g3):
    c = lax.axis_index("c")
    s = lax.axis_index("s")
    wid = c * NS + s
    si = (si0, si1, si2, si3)
    di = (di0, di1, di2, di3)
    rows = (r0, r1, r2, r3)
    gs = (g0, g1, g2, g3)

    for hs_hbm, src_hbm, dst_hbm, out_hbm, epw, nch in (
            (hsp_hbm, srcp_hbm, dstp_hbm, outp_hbm, EPW_P, NCH_P),
            (hsr_hbm, srcr_hbm, dstr_hbm, outr_hbm, EPW_R, NCH_R)):
        base = wid * epw
        pltpu.sync_copy(zeros_hbm, acc_sh.at[pl.ds(s * ROWS_PT, ROWS_PT)])
        plsc.subcore_barrier()

        def stage_and_fire(j, b):
            pltpu.sync_copy(src_hbm.at[pl.ds(base + j * CH, CH)], si[b])
            pltpu.sync_copy(dst_hbm.at[pl.ds(base + j * CH, CH)], di[b])
            pltpu.async_copy(hs_hbm.at[si[b]], rows[b], gs[b])

        def finish(b):
            pltpu.make_async_copy(hs_hbm.at[si[b]], rows[b], gs[b]).wait()
            pltpu.sync_copy(rows[b], acc_sh.at[di[b]], add=True)

        for b in range(NB):
            stage_and_fire(b, b)

        @pl.loop(0, nch - NB, step=NB)
        def _grp(jj):
            for b in range(NB):
                finish(b)
                stage_and_fire(jj + b + NB, b)

        for b in range(NB):
            finish(b)

        plsc.subcore_barrier()
        pltpu.sync_copy(acc_sh.at[pl.ds(s * ROWS_PT, ROWS_PT)],
                        out_hbm.at[c, pl.ds(s * ROWS_PT, ROWS_PT)])
        plsc.subcore_barrier()


def _sc_edges(hs_p, src_p, dst_p, hs_r, src_r, dst_r):
    zeros = jnp.zeros((ROWS_PT, D), jnp.float32)
    idx_bufs = [pltpu.VMEM((CH,), jnp.int32) for _ in range(2 * NB)]
    rows_bufs = [pltpu.VMEM((CH, D), jnp.float32) for _ in range(NB)]
    sems = [pltpu.SemaphoreType.DMA for _ in range(NB)]
    k = pl.kernel(
        _edges_body,
        mesh=_mesh(),
        compiler_params=pltpu.CompilerParams(
            needs_layout_passes=False, use_tc_tiling_on_sc=False),
        out_type=(jax.ShapeDtypeStruct((NC, ACC, D), jnp.float32),
                  jax.ShapeDtypeStruct((NC, ACC, D), jnp.float32)),
        scratch_types=[
            *idx_bufs,
            *rows_bufs,
            pltpu.MemorySpace.VMEM_SHARED((ACC, D), jnp.float32),
            *sems,
        ],
    )
    return k(hs_p, src_p, dst_p, hs_r, src_r, dst_r, zeros)


def _final_body(accp_ref, hsp_ref, degp_ref, ohp_ref, bp_ref,
                accr_ref, hsr_ref, degr_ref, ohr_ref, br_ref,
                lw_ref, lb_ref, out_ref, poolp, poolr):
    i = pl.program_id(0)

    @pl.when(i == 0)
    def _():
        poolp[...] = jnp.zeros_like(poolp)
        poolr[...] = jnp.zeros_like(poolr)

    def branch(acc_ref, hs_ref, deg_ref, oh_ref, b_ref, pool_ref):
        a = acc_ref[...]
        hs = hs_ref[...]
        dinv = lax.rsqrt(deg_ref[...])
        node = (a[0] + a[1] + hs) * dinv + b_ref[...]
        node = jnp.maximum(node, 0.0)
        aug = jnp.concatenate([node, jnp.ones_like(node)], axis=1)
        pool_ref[...] += lax.dot_general(
            oh_ref[...], aug, (((0,), (0,)), ((), ())),
            preferred_element_type=jnp.float32)

    branch(accp_ref, hsp_ref, degp_ref, ohp_ref, bp_ref, poolp)
    branch(accr_ref, hsr_ref, degr_ref, ohr_ref, br_ref, poolr)

    @pl.when(i == pl.num_programs(0) - 1)
    def _():
        pp = poolp[...]
        pr = poolr[...]
        mp = pp[:, :D] / jnp.maximum(pp[:, D:D + 1], 1.0)
        mr = pr[:, :D] / jnp.maximum(pr[:, D:D + 1], 1.0)
        feat = jnp.concatenate([mp, mr], axis=1)
        out_ref[...] = (jnp.dot(feat, lw_ref[...],
                                preferred_element_type=jnp.float32)
                        + lb_ref[...])


def _tc_final(accp, hs_p, degp, ohp, b_p, accr, hs_r, degr, ohr, b_r,
              lin_W, lin_b):
    return pl.pallas_call(
        _final_body,
        grid=(N // BLK,),
        in_specs=[
            pl.BlockSpec((NC, BLK, D), lambda i: (0, i, 0)),
            pl.BlockSpec((BLK, D), lambda i: (i, 0)),
            pl.BlockSpec((BLK, 1), lambda i: (i, 0)),
            pl.BlockSpec((BLK, G), lambda i: (i, 0)),
            pl.BlockSpec((1, D), lambda i: (0, 0)),
            pl.BlockSpec((NC, BLK, D), lambda i: (0, i, 0)),
            pl.BlockSpec((BLK, D), lambda i: (i, 0)),
            pl.BlockSpec((BLK, 1), lambda i: (i, 0)),
            pl.BlockSpec((BLK, G), lambda i: (i, 0)),
            pl.BlockSpec((1, D), lambda i: (0, 0)),
            pl.BlockSpec((D * 2, 2), lambda i: (0, 0)),
            pl.BlockSpec((1, 2), lambda i: (0, 0)),
        ],
        out_specs=pl.BlockSpec((G, 2), lambda i: (0, 0)),
        out_shape=jax.ShapeDtypeStruct((G, 2), jnp.float32),
        scratch_shapes=[
            pltpu.VMEM((G, 2 * D), jnp.float32),
            pltpu.VMEM((G, 2 * D), jnp.float32),
        ],
    )(accp, hs_p, degp, ohp, b_p, accr, hs_r, degr, ohr, b_r, lin_W, lin_b)


def _deg_to_col(deg):
    return jnp.concatenate(
        [deg, jnp.ones((ACC - BINS_H,), jnp.float32)]).reshape(ACC, 1)


def kernel(p_node_feat, p_edge_index, p_batch, r_node_feat, r_edge_index,
           r_batch, W_p, b_p, W_r, b_r, lin_W, lin_b):
    src_p, dst_p = _pad_edges(p_edge_index, NS * EPB_P)
    src_r, dst_r = _pad_edges(r_edge_index, NS * EPB_R)

    degp, degr = _sc_deg(dst_p, dst_r)
    degp_col = _deg_to_col(degp)
    degr_col = _deg_to_col(degr)

    hs_p = _tc_matmul_scale(p_node_feat.astype(jnp.float32), W_p, degp_col)
    hs_r = _tc_matmul_scale(r_node_feat.astype(jnp.float32), W_r, degr_col)

    accp, accr = _sc_edges(hs_p, src_p, dst_p, hs_r, src_r, dst_r)

    gids = jnp.arange(G, dtype=jnp.int32)
    ohp = (p_batch.astype(jnp.int32)[:, None] == gids[None, :]).astype(jnp.float32)
    ohr = (r_batch.astype(jnp.int32)[:, None] == gids[None, :]).astype(jnp.float32)

    return _tc_final(accp, hs_p, degp_col, ohp, b_p.reshape(1, D),
                     accr, hs_r, degr_col, ohr, b_r.reshape(1, D),
                     lin_W, lin_b.reshape(1, 2))


# speedup vs baseline: 1.9258x; 1.7554x over previous
import functools

import jax
import jax.numpy as jnp
from jax import lax
from jax.experimental import pallas as pl
from jax.experimental.pallas import tpu as pltpu, tpu_sc as plsc

N = 10000
D = 64
G = 256
NC = 2
NS = 16
NW = NC * NS
BINS_H = 10240
ACC = 12000
SENT = N
CH = 128
BLK = 1000
ROWS_PT = ACC // NS
BPT = BINS_H // NW

EPB_P = 4096
EPB_R = 40960
EPW_P = 2048
EPW_R = 20480
NCH_P = EPW_P // CH
NCH_R = EPW_R // CH
KG = 2

def _mesh():
    return plsc.VectorSubcoreMesh(core_axis_name="c", subcore_axis_name="s")


_sc_params = pltpu.CompilerParams(needs_layout_passes=False)


def _pad_edges(ei, e_pad):
    src = ei[0].astype(jnp.int32)
    dst = ei[1].astype(jnp.int32)
    e = src.shape[0]
    src = jnp.concatenate([src, jnp.zeros((e_pad - e,), jnp.int32)])
    dst = jnp.concatenate([dst, jnp.full((e_pad - e,), SENT, jnp.int32)])
    return src, dst


def _deg_body(dstp_hbm, dstr_hbm, outp_hbm, outr_hbm,
              idx_v, hist_v, row_v, deg_v, hists_sh):
    c = lax.axis_index("c")
    s = lax.axis_index("s")
    ones = jnp.ones((16,), jnp.float32)
    zeros16 = jnp.zeros((16,), jnp.float32)
    sumbase = (c * NS + s) * BPT

    for dst_hbm, out_hbm, epb in ((dstp_hbm, outp_hbm, EPB_P),
                                  (dstr_hbm, outr_hbm, EPB_R)):
        def zb(i, _):
            hist_v[pl.ds(i * 16, 16)] = zeros16
            return ()
        lax.fori_loop(0, BINS_H // 16, zb, ())
        pltpu.sync_copy(dst_hbm.at[pl.ds(s * epb, epb)], idx_v.at[pl.ds(0, epb)])
        def hb(i, _):
            idx = idx_v[pl.ds(i * 16, 16)]
            plsc.addupdate_scatter(hist_v, [idx], ones)
            return ()
        lax.fori_loop(0, epb // 16, hb, ())
        pltpu.sync_copy(hist_v, hists_sh.at[pl.ds(s * BINS_H, BINS_H)])
        plsc.subcore_barrier()
        def db(i, _):
            deg_v[pl.ds(i * 16, 16)] = ones
            return ()
        lax.fori_loop(0, BPT // 16, db, ())
        for t in range(NS):
            pltpu.sync_copy(hists_sh.at[pl.ds(t * BINS_H + sumbase, BPT)], row_v)
            def ab(i, _):
                deg_v[pl.ds(i * 16, 16)] += row_v[pl.ds(i * 16, 16)]
                return ()
            lax.fori_loop(0, BPT // 16, ab, ())
        pltpu.sync_copy(deg_v, out_hbm.at[pl.ds(sumbase, BPT)])
        plsc.subcore_barrier()


def _sc_deg(dst_p, dst_r):
    k = pl.kernel(
        _deg_body,
        mesh=_mesh(),
        compiler_params=_sc_params,
        out_type=(jax.ShapeDtypeStruct((BINS_H,), jnp.float32),
                  jax.ShapeDtypeStruct((BINS_H,), jnp.float32)),
        scratch_types=[
            pltpu.VMEM((EPB_R,), jnp.int32),
            pltpu.VMEM((BINS_H,), jnp.float32),
            pltpu.VMEM((BPT,), jnp.float32),
            pltpu.VMEM((BPT,), jnp.float32),
            pltpu.MemorySpace.VMEM_SHARED((NS * BINS_H,), jnp.float32),
        ],
    )
    return k(dst_p, dst_r)


def _mm_body(x_ref, w_ref, deg_ref, o_ref):
    h = jnp.dot(x_ref[...], w_ref[...], preferred_element_type=jnp.float32)
    o_ref[...] = h * lax.rsqrt(deg_ref[...])


def _tc_matmul_scale(x, w, deg_col):
    f = x.shape[1]
    return pl.pallas_call(
        _mm_body,
        grid=(N // BLK,),
        in_specs=[
            pl.BlockSpec((BLK, f), lambda i: (i, 0)),
            pl.BlockSpec((f, D), lambda i: (0, 0)),
            pl.BlockSpec((BLK, 1), lambda i: (i, 0)),
        ],
        out_specs=pl.BlockSpec((BLK, D), lambda i: (i, 0)),
        out_shape=jax.ShapeDtypeStruct((N, D), jnp.float32),
    )(x, w, deg_col)


NB = 4


def _edges_body(hsp_hbm, srcp_hbm, dstp_hbm, hsr_hbm, srcr_hbm, dstr_hbm,
                zeros_hbm, outp_hbm, outr_hbm,
                si0, si1, si2, si3, di0, di1, di2, di3,
                r0, r1, r2, r3, tab_sh, acc_sh, g0, g1, g2, g3):
    c = lax.axis_index("c")
    s = lax.axis_index("s")
    wid = c * NS + s
    si = (si0, si1, si2, si3)
    di = (di0, di1, di2, di3)
    rows = (r0, r1, r2, r3)
    gs = (g0, g1, g2, g3)

    for hs_hbm, src_hbm, dst_hbm, out_hbm, epw, nch in (
            (hsp_hbm, srcp_hbm, dstp_hbm, outp_hbm, EPW_P, NCH_P),
            (hsr_hbm, srcr_hbm, dstr_hbm, outr_hbm, EPW_R, NCH_R)):
        base = wid * epw
        pltpu.sync_copy(hs_hbm.at[pl.ds(s * (N // NS), N // NS)],
                        tab_sh.at[pl.ds(s * (N // NS), N // NS)])
        pltpu.sync_copy(zeros_hbm, acc_sh.at[pl.ds(s * ROWS_PT, ROWS_PT)])
        plsc.subcore_barrier()

        def stage_and_fire(j, b):
            pltpu.sync_copy(src_hbm.at[pl.ds(base + j * CH, CH)], si[b])
            pltpu.sync_copy(dst_hbm.at[pl.ds(base + j * CH, CH)], di[b])
            pltpu.async_copy(tab_sh.at[si[b]], rows[b], gs[b])

        def finish(b):
            pltpu.make_async_copy(tab_sh.at[si[b]], rows[b], gs[b]).wait()
            pltpu.sync_copy(rows[b], acc_sh.at[di[b]], add=True)

        for b in range(NB):
            stage_and_fire(b, b)

        @pl.loop(0, nch - NB, step=NB)
        def _grp(jj):
            for b in range(NB):
                finish(b)
                stage_and_fire(jj + b + NB, b)

        for b in range(NB):
            finish(b)

        plsc.subcore_barrier()
        pltpu.sync_copy(acc_sh.at[pl.ds(s * ROWS_PT, ROWS_PT)],
                        out_hbm.at[c, pl.ds(s * ROWS_PT, ROWS_PT)])
        plsc.subcore_barrier()


def _sc_edges(hs_p, src_p, dst_p, hs_r, src_r, dst_r):
    zeros = jnp.zeros((ROWS_PT, D), jnp.float32)
    idx_bufs = [pltpu.VMEM((CH,), jnp.int32) for _ in range(2 * NB)]
    rows_bufs = [pltpu.VMEM((CH, D), jnp.float32) for _ in range(NB)]
    sems = [pltpu.SemaphoreType.DMA for _ in range(NB)]
    k = pl.kernel(
        _edges_body,
        mesh=_mesh(),
        compiler_params=pltpu.CompilerParams(
            needs_layout_passes=False, use_tc_tiling_on_sc=False),
        out_type=(jax.ShapeDtypeStruct((NC, ACC, D), jnp.float32),
                  jax.ShapeDtypeStruct((NC, ACC, D), jnp.float32)),
        scratch_types=[
            *idx_bufs,
            *rows_bufs,
            pltpu.MemorySpace.VMEM_SHARED((N, D), jnp.float32),
            pltpu.MemorySpace.VMEM_SHARED((ACC, D), jnp.float32),
            *sems,
        ],
    )
    return k(hs_p, src_p, dst_p, hs_r, src_r, dst_r, zeros)


def _final_body(accp_ref, hsp_ref, degp_ref, ohp_ref, bp_ref,
                accr_ref, hsr_ref, degr_ref, ohr_ref, br_ref,
                lw_ref, lb_ref, out_ref, poolp, poolr):
    i = pl.program_id(0)

    @pl.when(i == 0)
    def _():
        poolp[...] = jnp.zeros_like(poolp)
        poolr[...] = jnp.zeros_like(poolr)

    def branch(acc_ref, hs_ref, deg_ref, oh_ref, b_ref, pool_ref):
        a = acc_ref[...]
        hs = hs_ref[...]
        dinv = lax.rsqrt(deg_ref[...])
        node = (a[0] + a[1] + hs) * dinv + b_ref[...]
        node = jnp.maximum(node, 0.0)
        aug = jnp.concatenate([node, jnp.ones_like(node)], axis=1)
        pool_ref[...] += lax.dot_general(
            oh_ref[...], aug, (((0,), (0,)), ((), ())),
            preferred_element_type=jnp.float32)

    branch(accp_ref, hsp_ref, degp_ref, ohp_ref, bp_ref, poolp)
    branch(accr_ref, hsr_ref, degr_ref, ohr_ref, br_ref, poolr)

    @pl.when(i == pl.num_programs(0) - 1)
    def _():
        pp = poolp[...]
        pr = poolr[...]
        mp = pp[:, :D] / jnp.maximum(pp[:, D:D + 1], 1.0)
        mr = pr[:, :D] / jnp.maximum(pr[:, D:D + 1], 1.0)
        feat = jnp.concatenate([mp, mr], axis=1)
        out_ref[...] = (jnp.dot(feat, lw_ref[...],
                                preferred_element_type=jnp.float32)
                        + lb_ref[...])


def _tc_final(accp, hs_p, degp, ohp, b_p, accr, hs_r, degr, ohr, b_r,
              lin_W, lin_b):
    return pl.pallas_call(
        _final_body,
        grid=(N // BLK,),
        in_specs=[
            pl.BlockSpec((NC, BLK, D), lambda i: (0, i, 0)),
            pl.BlockSpec((BLK, D), lambda i: (i, 0)),
            pl.BlockSpec((BLK, 1), lambda i: (i, 0)),
            pl.BlockSpec((BLK, G), lambda i: (i, 0)),
            pl.BlockSpec((1, D), lambda i: (0, 0)),
            pl.BlockSpec((NC, BLK, D), lambda i: (0, i, 0)),
            pl.BlockSpec((BLK, D), lambda i: (i, 0)),
            pl.BlockSpec((BLK, 1), lambda i: (i, 0)),
            pl.BlockSpec((BLK, G), lambda i: (i, 0)),
            pl.BlockSpec((1, D), lambda i: (0, 0)),
            pl.BlockSpec((D * 2, 2), lambda i: (0, 0)),
            pl.BlockSpec((1, 2), lambda i: (0, 0)),
        ],
        out_specs=pl.BlockSpec((G, 2), lambda i: (0, 0)),
        out_shape=jax.ShapeDtypeStruct((G, 2), jnp.float32),
        scratch_shapes=[
            pltpu.VMEM((G, 2 * D), jnp.float32),
            pltpu.VMEM((G, 2 * D), jnp.float32),
        ],
    )(accp, hs_p, degp, ohp, b_p, accr, hs_r, degr, ohr, b_r, lin_W, lin_b)


def _deg_to_col(deg):
    return jnp.concatenate(
        [deg, jnp.ones((ACC - BINS_H,), jnp.float32)]).reshape(ACC, 1)


def kernel(p_node_feat, p_edge_index, p_batch, r_node_feat, r_edge_index,
           r_batch, W_p, b_p, W_r, b_r, lin_W, lin_b):
    src_p, dst_p = _pad_edges(p_edge_index, NS * EPB_P)
    src_r, dst_r = _pad_edges(r_edge_index, NS * EPB_R)

    degp, degr = _sc_deg(dst_p, dst_r)
    degp_col = _deg_to_col(degp)
    degr_col = _deg_to_col(degr)

    hs_p = _tc_matmul_scale(p_node_feat.astype(jnp.float32), W_p, degp_col)
    hs_r = _tc_matmul_scale(r_node_feat.astype(jnp.float32), W_r, degr_col)

    accp, accr = _sc_edges(hs_p, src_p, dst_p, hs_r, src_r, dst_r)

    gids = jnp.arange(G, dtype=jnp.int32)
    ohp = (p_batch.astype(jnp.int32)[:, None] == gids[None, :]).astype(jnp.float32)
    ohr = (r_batch.astype(jnp.int32)[:, None] == gids[None, :]).astype(jnp.float32)

    return _tc_final(accp, hs_p, degp_col, ohp, b_p.reshape(1, D),
                     accr, hs_r, degr_col, ohr, b_r.reshape(1, D),
                     lin_W, lin_b.reshape(1, 2))


# speedup vs baseline: 2.0377x; 1.0581x over previous
import functools

import jax
import jax.numpy as jnp
from jax import lax
from jax.experimental import pallas as pl
from jax.experimental.pallas import tpu as pltpu, tpu_sc as plsc

N = 10000
D = 64
G = 256
NC = 2
NS = 16
NW = NC * NS
BINS_H = 10240
ACC = 12000
SENT = N
CH = 128
BLK = 1000
ROWS_PT = ACC // NS
BPT = BINS_H // NW

EPB_P = 4096
EPB_R = 40960
EPW_P = 2048
EPW_R = 20480
NCH_P = EPW_P // CH
NCH_R = EPW_R // CH
KG = 2

def _mesh():
    return plsc.VectorSubcoreMesh(core_axis_name="c", subcore_axis_name="s")


_sc_params = pltpu.CompilerParams(needs_layout_passes=False)


def _pad_edges(ei, e_pad):
    src = ei[0].astype(jnp.int32)
    dst = ei[1].astype(jnp.int32)
    e = src.shape[0]
    src = jnp.concatenate([src, jnp.zeros((e_pad - e,), jnp.int32)])
    dst = jnp.concatenate([dst, jnp.full((e_pad - e,), SENT, jnp.int32)])
    return src, dst


def _deg_body(dstp_hbm, dstr_hbm, outp_hbm, outr_hbm,
              idx_v, hist_v, row_v, deg_v, hists_sh):
    c = lax.axis_index("c")
    s = lax.axis_index("s")
    ones = jnp.ones((16,), jnp.float32)
    zeros16 = jnp.zeros((16,), jnp.float32)
    sumbase = (c * NS + s) * BPT

    for dst_hbm, out_hbm, epb in ((dstp_hbm, outp_hbm, EPB_P),
                                  (dstr_hbm, outr_hbm, EPB_R)):
        def zb(i, _):
            hist_v[pl.ds(i * 16, 16)] = zeros16
            return ()
        lax.fori_loop(0, BINS_H // 16, zb, (), unroll=8)
        pltpu.sync_copy(dst_hbm.at[pl.ds(s * epb, epb)], idx_v.at[pl.ds(0, epb)])
        def hb(i, _):
            idx = idx_v[pl.ds(i * 16, 16)]
            plsc.addupdate_scatter(hist_v, [idx], ones)
            return ()
        lax.fori_loop(0, epb // 16, hb, (), unroll=8)
        pltpu.sync_copy(hist_v, hists_sh.at[pl.ds(s * BINS_H, BINS_H)])
        plsc.subcore_barrier()
        def db(i, _):
            deg_v[pl.ds(i * 16, 16)] = ones
            return ()
        lax.fori_loop(0, BPT // 16, db, ())
        for t in range(NS):
            pltpu.sync_copy(hists_sh.at[pl.ds(t * BINS_H + sumbase, BPT)], row_v)
            def ab(i, _):
                deg_v[pl.ds(i * 16, 16)] += row_v[pl.ds(i * 16, 16)]
                return ()
            lax.fori_loop(0, BPT // 16, ab, (), unroll=8)
        pltpu.sync_copy(deg_v, out_hbm.at[pl.ds(sumbase, BPT)])
        plsc.subcore_barrier()


def _sc_deg(dst_p, dst_r):
    k = pl.kernel(
        _deg_body,
        mesh=_mesh(),
        compiler_params=_sc_params,
        out_type=(jax.ShapeDtypeStruct((BINS_H,), jnp.float32),
                  jax.ShapeDtypeStruct((BINS_H,), jnp.float32)),
        scratch_types=[
            pltpu.VMEM((EPB_R,), jnp.int32),
            pltpu.VMEM((BINS_H,), jnp.float32),
            pltpu.VMEM((BPT,), jnp.float32),
            pltpu.VMEM((BPT,), jnp.float32),
            pltpu.MemorySpace.VMEM_SHARED((NS * BINS_H,), jnp.float32),
        ],
    )
    return k(dst_p, dst_r)


def _mm_body(x_ref, w_ref, o_ref):
    o_ref[...] = jnp.dot(x_ref[...], w_ref[...],
                         preferred_element_type=jnp.float32)


def _tc_matmul(x, w):
    f = x.shape[1]
    return pl.pallas_call(
        _mm_body,
        grid=(N // BLK,),
        in_specs=[
            pl.BlockSpec((BLK, f), lambda i: (i, 0)),
            pl.BlockSpec((f, D), lambda i: (0, 0)),
        ],
        out_specs=pl.BlockSpec((BLK, D), lambda i: (i, 0)),
        out_shape=jax.ShapeDtypeStruct((N, D), jnp.float32),
    )(x, w)


def _scale_body(hp_ref, degp_ref, hr_ref, degr_ref, op_ref, or_ref):
    op_ref[...] = hp_ref[...] * lax.rsqrt(degp_ref[...])
    or_ref[...] = hr_ref[...] * lax.rsqrt(degr_ref[...])


def _tc_scale(h_p, degp_col, h_r, degr_col):
    return pl.pallas_call(
        _scale_body,
        grid=(N // BLK,),
        in_specs=[
            pl.BlockSpec((BLK, D), lambda i: (i, 0)),
            pl.BlockSpec((BLK, 1), lambda i: (i, 0)),
            pl.BlockSpec((BLK, D), lambda i: (i, 0)),
            pl.BlockSpec((BLK, 1), lambda i: (i, 0)),
        ],
        out_specs=[
            pl.BlockSpec((BLK, D), lambda i: (i, 0)),
            pl.BlockSpec((BLK, D), lambda i: (i, 0)),
        ],
        out_shape=[jax.ShapeDtypeStruct((N, D), jnp.float32),
                   jax.ShapeDtypeStruct((N, D), jnp.float32)],
    )(h_p, degp_col, h_r, degr_col)


NB = 4


def _edges_body(hsp_hbm, srcp_hbm, dstp_hbm, hsr_hbm, srcr_hbm, dstr_hbm,
                zeros_hbm, outp_hbm, outr_hbm,
                si0, si1, si2, si3, di0, di1, di2, di3,
                r0, r1, r2, r3, tab_sh, acc_sh, g0, g1, g2, g3):
    c = lax.axis_index("c")
    s = lax.axis_index("s")
    wid = c * NS + s
    si = (si0, si1, si2, si3)
    di = (di0, di1, di2, di3)
    rows = (r0, r1, r2, r3)
    gs = (g0, g1, g2, g3)

    for hs_hbm, src_hbm, dst_hbm, out_hbm, epw, nch in (
            (hsp_hbm, srcp_hbm, dstp_hbm, outp_hbm, EPW_P, NCH_P),
            (hsr_hbm, srcr_hbm, dstr_hbm, outr_hbm, EPW_R, NCH_R)):
        base = wid * epw
        pltpu.sync_copy(hs_hbm.at[pl.ds(s * (N // NS), N // NS)],
                        tab_sh.at[pl.ds(s * (N // NS), N // NS)])
        pltpu.sync_copy(zeros_hbm, acc_sh.at[pl.ds(s * ROWS_PT, ROWS_PT)])
        plsc.subcore_barrier()

        def stage_and_fire(j, b):
            pltpu.sync_copy(src_hbm.at[pl.ds(base + j * CH, CH)], si[b])
            pltpu.sync_copy(dst_hbm.at[pl.ds(base + j * CH, CH)], di[b])
            pltpu.async_copy(tab_sh.at[si[b]], rows[b], gs[b])

        def finish(b):
            pltpu.make_async_copy(tab_sh.at[si[b]], rows[b], gs[b]).wait()
            pltpu.sync_copy(rows[b], acc_sh.at[di[b]], add=True)

        for b in range(NB):
            stage_and_fire(b, b)

        @pl.loop(0, nch - NB, step=NB)
        def _grp(jj):
            for b in range(NB):
                finish(b)
                stage_and_fire(jj + b + NB, b)

        for b in range(NB):
            finish(b)

        plsc.subcore_barrier()
        pltpu.sync_copy(acc_sh.at[pl.ds(s * ROWS_PT, ROWS_PT)],
                        out_hbm.at[c, pl.ds(s * ROWS_PT, ROWS_PT)])
        plsc.subcore_barrier()


def _sc_edges(hs_p, src_p, dst_p, hs_r, src_r, dst_r):
    zeros = jnp.zeros((ROWS_PT, D), jnp.float32)
    idx_bufs = [pltpu.VMEM((CH,), jnp.int32) for _ in range(2 * NB)]
    rows_bufs = [pltpu.VMEM((CH, D), jnp.float32) for _ in range(NB)]
    sems = [pltpu.SemaphoreType.DMA for _ in range(NB)]
    k = pl.kernel(
        _edges_body,
        mesh=_mesh(),
        compiler_params=pltpu.CompilerParams(
            needs_layout_passes=False, use_tc_tiling_on_sc=False),
        out_type=(jax.ShapeDtypeStruct((NC, ACC, D), jnp.float32),
                  jax.ShapeDtypeStruct((NC, ACC, D), jnp.float32)),
        scratch_types=[
            *idx_bufs,
            *rows_bufs,
            pltpu.MemorySpace.VMEM_SHARED((N, D), jnp.float32),
            pltpu.MemorySpace.VMEM_SHARED((ACC, D), jnp.float32),
            *sems,
        ],
    )
    return k(hs_p, src_p, dst_p, hs_r, src_r, dst_r, zeros)


def _final_body(accp_ref, hsp_ref, degp_ref, ohp_ref, bp_ref,
                accr_ref, hsr_ref, degr_ref, ohr_ref, br_ref,
                lw_ref, lb_ref, out_ref, poolp, poolr):
    i = pl.program_id(0)

    @pl.when(i == 0)
    def _():
        poolp[...] = jnp.zeros_like(poolp)
        poolr[...] = jnp.zeros_like(poolr)

    def branch(acc_ref, hs_ref, deg_ref, oh_ref, b_ref, pool_ref):
        a = acc_ref[...]
        hs = hs_ref[...]
        dinv = lax.rsqrt(deg_ref[...])
        node = (a[0] + a[1] + hs) * dinv + b_ref[...]
        node = jnp.maximum(node, 0.0)
        aug = jnp.concatenate([node, jnp.ones_like(node)], axis=1)
        pool_ref[...] += lax.dot_general(
            oh_ref[...], aug, (((0,), (0,)), ((), ())),
            preferred_element_type=jnp.float32)

    branch(accp_ref, hsp_ref, degp_ref, ohp_ref, bp_ref, poolp)
    branch(accr_ref, hsr_ref, degr_ref, ohr_ref, br_ref, poolr)

    @pl.when(i == pl.num_programs(0) - 1)
    def _():
        pp = poolp[...]
        pr = poolr[...]
        mp = pp[:, :D] / jnp.maximum(pp[:, D:D + 1], 1.0)
        mr = pr[:, :D] / jnp.maximum(pr[:, D:D + 1], 1.0)
        feat = jnp.concatenate([mp, mr], axis=1)
        out_ref[...] = (jnp.dot(feat, lw_ref[...],
                                preferred_element_type=jnp.float32)
                        + lb_ref[...])


def _tc_final(accp, hs_p, degp, ohp, b_p, accr, hs_r, degr, ohr, b_r,
              lin_W, lin_b):
    return pl.pallas_call(
        _final_body,
        grid=(N // BLK,),
        in_specs=[
            pl.BlockSpec((NC, BLK, D), lambda i: (0, i, 0)),
            pl.BlockSpec((BLK, D), lambda i: (i, 0)),
            pl.BlockSpec((BLK, 1), lambda i: (i, 0)),
            pl.BlockSpec((BLK, G), lambda i: (i, 0)),
            pl.BlockSpec((1, D), lambda i: (0, 0)),
            pl.BlockSpec((NC, BLK, D), lambda i: (0, i, 0)),
            pl.BlockSpec((BLK, D), lambda i: (i, 0)),
            pl.BlockSpec((BLK, 1), lambda i: (i, 0)),
            pl.BlockSpec((BLK, G), lambda i: (i, 0)),
            pl.BlockSpec((1, D), lambda i: (0, 0)),
            pl.BlockSpec((D * 2, 2), lambda i: (0, 0)),
            pl.BlockSpec((1, 2), lambda i: (0, 0)),
        ],
        out_specs=pl.BlockSpec((G, 2), lambda i: (0, 0)),
        out_shape=jax.ShapeDtypeStruct((G, 2), jnp.float32),
        scratch_shapes=[
            pltpu.VMEM((G, 2 * D), jnp.float32),
            pltpu.VMEM((G, 2 * D), jnp.float32),
        ],
    )(accp, hs_p, degp, ohp, b_p, accr, hs_r, degr, ohr, b_r, lin_W, lin_b)


def _deg_to_col(deg):
    return jnp.concatenate(
        [deg, jnp.ones((ACC - BINS_H,), jnp.float32)]).reshape(ACC, 1)


def kernel(p_node_feat, p_edge_index, p_batch, r_node_feat, r_edge_index,
           r_batch, W_p, b_p, W_r, b_r, lin_W, lin_b):
    src_p, dst_p = _pad_edges(p_edge_index, NS * EPB_P)
    src_r, dst_r = _pad_edges(r_edge_index, NS * EPB_R)

    degp, degr = _sc_deg(dst_p, dst_r)
    degp_col = _deg_to_col(degp)
    degr_col = _deg_to_col(degr)

    h_p = _tc_matmul(p_node_feat.astype(jnp.float32), W_p)
    h_r = _tc_matmul(r_node_feat.astype(jnp.float32), W_r)
    hs_p, hs_r = _tc_scale(h_p, degp_col, h_r, degr_col)

    accp, accr = _sc_edges(hs_p, src_p, dst_p, hs_r, src_r, dst_r)

    gids = jnp.arange(G, dtype=jnp.int32)
    ohp = (p_batch.astype(jnp.int32)[:, None] == gids[None, :]).astype(jnp.float32)
    ohr = (r_batch.astype(jnp.int32)[:, None] == gids[None, :]).astype(jnp.float32)

    return _tc_final(accp, hs_p, degp_col, ohp, b_p.reshape(1, D),
                     accr, hs_r, degr_col, ohr, b_r.reshape(1, D),
                     lin_W, lin_b.reshape(1, 2))


# speedup vs baseline: 2.0616x; 1.0117x over previous
import functools

import jax
import jax.numpy as jnp
from jax import lax
from jax.experimental import pallas as pl
from jax.experimental.pallas import tpu as pltpu, tpu_sc as plsc

N = 10000
D = 64
G = 256
NC = 2
NS = 16
NW = NC * NS
BINS_H = 10240
ACC = 12000
SENT = N
CH = 128
BLK = 1000
ROWS_PT = ACC // NS
BPT = BINS_H // NW

EPB_P = 4096
EPB_R = 40960
EPW_P = 2048
EPW_R = 20480
NCH_P = EPW_P // CH
NCH_R = EPW_R // CH
KG = 2

def _mesh():
    return plsc.VectorSubcoreMesh(core_axis_name="c", subcore_axis_name="s")


_sc_params = pltpu.CompilerParams(needs_layout_passes=False)


def _pad_edges(ei, e_pad):
    src = ei[0].astype(jnp.int32)
    dst = ei[1].astype(jnp.int32)
    e = src.shape[0]
    src = jnp.concatenate([src, jnp.zeros((e_pad - e,), jnp.int32)])
    dst = jnp.concatenate([dst, jnp.full((e_pad - e,), SENT, jnp.int32)])
    return src, dst


def _deg_body(dstp_hbm, dstr_hbm, outp_hbm, outr_hbm,
              idx_v, hist_v, row_v, deg_v, hists_sh):
    c = lax.axis_index("c")
    s = lax.axis_index("s")
    ones = jnp.ones((16,), jnp.float32)
    zeros16 = jnp.zeros((16,), jnp.float32)
    sumbase = (c * NS + s) * BPT

    for dst_hbm, out_hbm, epb in ((dstp_hbm, outp_hbm, EPB_P),
                                  (dstr_hbm, outr_hbm, EPB_R)):
        def zb(i, _):
            hist_v[pl.ds(i * 16, 16)] = zeros16
            return ()
        lax.fori_loop(0, BINS_H // 16, zb, (), unroll=8)
        pltpu.sync_copy(dst_hbm.at[pl.ds(s * epb, epb)], idx_v.at[pl.ds(0, epb)])
        def hb(i, _):
            idx = idx_v[pl.ds(i * 16, 16)]
            plsc.addupdate_scatter(hist_v, [idx], ones)
            return ()
        lax.fori_loop(0, epb // 16, hb, (), unroll=8)
        pltpu.sync_copy(hist_v, hists_sh.at[pl.ds(s * BINS_H, BINS_H)])
        plsc.subcore_barrier()
        def db(i, _):
            deg_v[pl.ds(i * 16, 16)] = ones
            return ()
        lax.fori_loop(0, BPT // 16, db, ())
        for t in range(NS):
            pltpu.sync_copy(hists_sh.at[pl.ds(t * BINS_H + sumbase, BPT)], row_v)
            def ab(i, _):
                deg_v[pl.ds(i * 16, 16)] += row_v[pl.ds(i * 16, 16)]
                return ()
            lax.fori_loop(0, BPT // 16, ab, (), unroll=8)
        pltpu.sync_copy(deg_v, out_hbm.at[pl.ds(sumbase, BPT)])
        plsc.subcore_barrier()


def _sc_deg(dst_p, dst_r):
    k = pl.kernel(
        _deg_body,
        mesh=_mesh(),
        compiler_params=_sc_params,
        out_type=(jax.ShapeDtypeStruct((BINS_H,), jnp.float32),
                  jax.ShapeDtypeStruct((BINS_H,), jnp.float32)),
        scratch_types=[
            pltpu.VMEM((EPB_R,), jnp.int32),
            pltpu.VMEM((BINS_H,), jnp.float32),
            pltpu.VMEM((BPT,), jnp.float32),
            pltpu.VMEM((BPT,), jnp.float32),
            pltpu.MemorySpace.VMEM_SHARED((NS * BINS_H,), jnp.float32),
        ],
    )
    return k(dst_p, dst_r)


def _mm_body(x_ref, w_ref, o_ref):
    o_ref[...] = jnp.dot(x_ref[...], w_ref[...],
                         preferred_element_type=jnp.float32)


def _tc_matmul(x, w):
    f = x.shape[1]
    return pl.pallas_call(
        _mm_body,
        grid=(N // BLK,),
        in_specs=[
            pl.BlockSpec((BLK, f), lambda i: (i, 0)),
            pl.BlockSpec((f, D), lambda i: (0, 0)),
        ],
        out_specs=pl.BlockSpec((BLK, D), lambda i: (i, 0)),
        out_shape=jax.ShapeDtypeStruct((N, D), jnp.float32),
    )(x, w)


def _scale_body(hp_ref, degp_ref, hr_ref, degr_ref, op_ref, or_ref):
    op_ref[...] = hp_ref[...] * lax.rsqrt(degp_ref[...])
    or_ref[...] = hr_ref[...] * lax.rsqrt(degr_ref[...])


def _tc_scale(h_p, degp_col, h_r, degr_col):
    return pl.pallas_call(
        _scale_body,
        grid=(N // BLK,),
        in_specs=[
            pl.BlockSpec((BLK, D), lambda i: (i, 0)),
            pl.BlockSpec((BLK, 1), lambda i: (i, 0)),
            pl.BlockSpec((BLK, D), lambda i: (i, 0)),
            pl.BlockSpec((BLK, 1), lambda i: (i, 0)),
        ],
        out_specs=[
            pl.BlockSpec((BLK, D), lambda i: (i, 0)),
            pl.BlockSpec((BLK, D), lambda i: (i, 0)),
        ],
        out_shape=[jax.ShapeDtypeStruct((N, D), jnp.float32),
                   jax.ShapeDtypeStruct((N, D), jnp.float32)],
    )(h_p, degp_col, h_r, degr_col)


NB = 4


def _edges_body(hsp_hbm, srcp_hbm, dstp_hbm, hsr_hbm, srcr_hbm, dstr_hbm,
                zeros_hbm, outp_hbm, outr_hbm,
                si0, si1, si2, si3, di0, di1, di2, di3,
                r0, r1, r2, r3, tab_sh, acc_sh, g0, g1, g2, g3):
    c = lax.axis_index("c")
    s = lax.axis_index("s")
    wid = c * NS + s
    si = (si0, si1, si2, si3)
    di = (di0, di1, di2, di3)
    rows = (r0, r1, r2, r3)
    gs = (g0, g1, g2, g3)

    for hs_hbm, src_hbm, dst_hbm, out_hbm, epw, nch in (
            (hsp_hbm, srcp_hbm, dstp_hbm, outp_hbm, EPW_P, NCH_P),
            (hsr_hbm, srcr_hbm, dstr_hbm, outr_hbm, EPW_R, NCH_R)):
        base = wid * epw
        pltpu.sync_copy(hs_hbm.at[pl.ds(s * (N // NS), N // NS)],
                        tab_sh.at[pl.ds(s * (N // NS), N // NS)])
        pltpu.sync_copy(zeros_hbm, acc_sh.at[pl.ds(s * ROWS_PT, ROWS_PT)])
        plsc.subcore_barrier()

        def stage_and_fire(j, b):
            pltpu.sync_copy(src_hbm.at[pl.ds(base + j * CH, CH)], si[b])
            pltpu.sync_copy(dst_hbm.at[pl.ds(base + j * CH, CH)], di[b])
            pltpu.async_copy(tab_sh.at[si[b]], rows[b], gs[b])

        def finish(b):
            pltpu.make_async_copy(tab_sh.at[si[b]], rows[b], gs[b]).wait()
            pltpu.sync_copy(rows[b], acc_sh.at[di[b]], add=True)

        for b in range(NB):
            stage_and_fire(b, b)

        @pl.loop(0, nch - NB, step=NB)
        def _grp(jj):
            for b in range(NB):
                finish(b)
                stage_and_fire(jj + b + NB, b)

        for b in range(NB):
            finish(b)

        plsc.subcore_barrier()
        pltpu.sync_copy(acc_sh.at[pl.ds(s * ROWS_PT, ROWS_PT)],
                        out_hbm.at[c, pl.ds(s * ROWS_PT, ROWS_PT)])
        plsc.subcore_barrier()


def _sc_edges(hs_p, src_p, dst_p, hs_r, src_r, dst_r):
    zeros = jnp.zeros((ROWS_PT, D), jnp.float32)
    idx_bufs = [pltpu.VMEM((CH,), jnp.int32) for _ in range(2 * NB)]
    rows_bufs = [pltpu.VMEM((CH, D), jnp.float32) for _ in range(NB)]
    sems = [pltpu.SemaphoreType.DMA for _ in range(NB)]
    k = pl.kernel(
        _edges_body,
        mesh=_mesh(),
        compiler_params=pltpu.CompilerParams(
            needs_layout_passes=False, use_tc_tiling_on_sc=False),
        out_type=(jax.ShapeDtypeStruct((NC, ACC, D), jnp.float32),
                  jax.ShapeDtypeStruct((NC, ACC, D), jnp.float32)),
        scratch_types=[
            *idx_bufs,
            *rows_bufs,
            pltpu.MemorySpace.VMEM_SHARED((N, D), jnp.float32),
            pltpu.MemorySpace.VMEM_SHARED((ACC, D), jnp.float32),
            *sems,
        ],
    )
    return k(hs_p, src_p, dst_p, hs_r, src_r, dst_r, zeros)


def _final_body(accp_ref, hsp_ref, degp_ref, batp_ref, bp_ref,
                accr_ref, hsr_ref, degr_ref, batr_ref, br_ref,
                lw_ref, lb_ref, out_ref, poolp, poolr):
    i = pl.program_id(0)

    @pl.when(i == 0)
    def _():
        poolp[...] = jnp.zeros_like(poolp)
        poolr[...] = jnp.zeros_like(poolr)

    def branch(acc_ref, hs_ref, deg_ref, bat_ref, b_ref, pool_ref):
        a = acc_ref[...]
        hs = hs_ref[...]
        dinv = lax.rsqrt(deg_ref[...])
        node = (a[0] + a[1] + hs) * dinv + b_ref[...]
        node = jnp.maximum(node, 0.0)
        aug = jnp.concatenate([node, jnp.ones_like(node)], axis=1)
        gids = lax.broadcasted_iota(jnp.int32, (1, G), 1)
        oh = (bat_ref[...] == gids).astype(jnp.float32)
        pool_ref[...] += lax.dot_general(
            oh, aug, (((0,), (0,)), ((), ())),
            preferred_element_type=jnp.float32)

    branch(accp_ref, hsp_ref, degp_ref, batp_ref, bp_ref, poolp)
    branch(accr_ref, hsr_ref, degr_ref, batr_ref, br_ref, poolr)

    @pl.when(i == pl.num_programs(0) - 1)
    def _():
        pp = poolp[...]
        pr = poolr[...]
        mp = pp[:, :D] / jnp.maximum(pp[:, D:D + 1], 1.0)
        mr = pr[:, :D] / jnp.maximum(pr[:, D:D + 1], 1.0)
        feat = jnp.concatenate([mp, mr], axis=1)
        out_ref[...] = (jnp.dot(feat, lw_ref[...],
                                preferred_element_type=jnp.float32)
                        + lb_ref[...])


def _tc_final(accp, hs_p, degp, batp, b_p, accr, hs_r, degr, batr, b_r,
              lin_W, lin_b):
    return pl.pallas_call(
        _final_body,
        grid=(N // BLK,),
        in_specs=[
            pl.BlockSpec((NC, BLK, D), lambda i: (0, i, 0)),
            pl.BlockSpec((BLK, D), lambda i: (i, 0)),
            pl.BlockSpec((BLK, 1), lambda i: (i, 0)),
            pl.BlockSpec((BLK, 1), lambda i: (i, 0)),
            pl.BlockSpec((1, D), lambda i: (0, 0)),
            pl.BlockSpec((NC, BLK, D), lambda i: (0, i, 0)),
            pl.BlockSpec((BLK, D), lambda i: (i, 0)),
            pl.BlockSpec((BLK, 1), lambda i: (i, 0)),
            pl.BlockSpec((BLK, 1), lambda i: (i, 0)),
            pl.BlockSpec((1, D), lambda i: (0, 0)),
            pl.BlockSpec((D * 2, 2), lambda i: (0, 0)),
            pl.BlockSpec((1, 2), lambda i: (0, 0)),
        ],
        out_specs=pl.BlockSpec((G, 2), lambda i: (0, 0)),
        out_shape=jax.ShapeDtypeStruct((G, 2), jnp.float32),
        scratch_shapes=[
            pltpu.VMEM((G, 2 * D), jnp.float32),
            pltpu.VMEM((G, 2 * D), jnp.float32),
        ],
    )(accp, hs_p, degp, batp, b_p, accr, hs_r, degr, batr, b_r, lin_W, lin_b)


def _deg_to_col(deg):
    return jnp.concatenate(
        [deg, jnp.ones((ACC - BINS_H,), jnp.float32)]).reshape(ACC, 1)


def kernel(p_node_feat, p_edge_index, p_batch, r_node_feat, r_edge_index,
           r_batch, W_p, b_p, W_r, b_r, lin_W, lin_b):
    src_p, dst_p = _pad_edges(p_edge_index, NS * EPB_P)
    src_r, dst_r = _pad_edges(r_edge_index, NS * EPB_R)

    degp, degr = _sc_deg(dst_p, dst_r)
    degp_col = _deg_to_col(degp)
    degr_col = _deg_to_col(degr)

    h_p = _tc_matmul(p_node_feat.astype(jnp.float32), W_p)
    h_r = _tc_matmul(r_node_feat.astype(jnp.float32), W_r)
    hs_p, hs_r = _tc_scale(h_p, degp_col, h_r, degr_col)

    accp, accr = _sc_edges(hs_p, src_p, dst_p, hs_r, src_r, dst_r)

    batp = p_batch.astype(jnp.int32).reshape(N, 1)
    batr = r_batch.astype(jnp.int32).reshape(N, 1)

    return _tc_final(accp, hs_p, degp_col, batp, b_p.reshape(1, D),
                     accr, hs_r, degr_col, batr, b_r.reshape(1, D),
                     lin_W, lin_b.reshape(1, 2))
